# baseline (speedup 1.0000x reference)
import functools

import jax
import jax.numpy as jnp
from jax import lax
from jax.experimental import pallas as pl
from jax.experimental.pallas import tpu as pltpu
from jax.experimental.pallas import tpu_sc as plsc

HALF = 16
RPG = 2
K = 8


def _tree_sum(vals):
    while len(vals) > 1:
        nxt = [vals[i] + vals[i + 1] for i in range(0, len(vals) - 1, 2)]
        if len(vals) % 2:
            nxt.append(vals[-1])
        vals = nxt
    return vals[0]


def _tr_body(in_ref, out_ref):
    eye = jnp.eye(32, dtype=jnp.float32)
    out_ref[...] = jnp.concatenate(
        [jax.lax.dot_general(in_ref[:, 128 * c:128 * (c + 1)], eye,
                             (((0,), (0,)), ((), ())))
         for c in range(4)], axis=1)


def kernel(ids, weights):
    B, L = ids.shape
    V, D = weights.shape
    info = plsc.get_sparse_core_info()
    nw = info.num_cores * info.num_subcores
    rows_w = B // nw
    idx_per_g = RPG * L
    ng = rows_w // RPG

    nbc = 512
    nb = (V + nbc - 1) // nbc
    wc = pl.pallas_call(
        _tr_body,
        grid=(nb,),
        in_specs=[pl.BlockSpec((D, nbc), lambda i: (0, i))],
        out_specs=pl.BlockSpec((nbc * D // 128, 128), lambda i: (i, 0)),
        out_shape=jax.ShapeDtypeStruct((nb * nbc * D // 128, 128),
                                       jnp.float32),
    )(weights.T)
    wl = wc.reshape(-1, D)
    idm = ((ids >> 9) << 9) + ((ids & 127) << 2) + ((ids >> 7) & 3)
    ids2 = idm.reshape(B // RPG, idx_per_g)

    mesh = plsc.VectorSubcoreMesh(core_axis_name="c", subcore_axis_name="s")

    @functools.partial(
        pl.kernel,
        mesh=mesh,
        compiler_params=pltpu.CompilerParams(use_tc_tiling_on_sc=False),
        out_type=jax.ShapeDtypeStruct((B * D // 128, 128), jnp.float32),
        scratch_types=[
            pltpu.VMEM((ng, idx_per_g), jnp.int32),
            pltpu.VMEM((K, idx_per_g, D), jnp.float32),
            pltpu.VMEM((rows_w * D // 128, 128), jnp.float32),
        ] + [pltpu.SemaphoreType.DMA] * K,
    )
    def run(ids_hbm, tab_hbm, out_hbm, ids_v, buf_v, out_v, *sems):
        wid = lax.axis_index("s") * info.num_cores + lax.axis_index("c")
        gbase = wid * ng
        orows_w = rows_w * D // 128
        pltpu.sync_copy(ids_hbm.at[pl.ds(gbase, ng)], ids_v)

        def fire(g, s):
            pltpu.async_copy(tab_hbm.at[ids_v.at[g]], buf_v.at[s], sems[s])

        def drain(g, s):
            pltpu.make_async_copy(
                tab_hbm.at[ids_v.at[g]], buf_v.at[s], sems[s]).wait()

        for s in range(K):
            fire(s, s)

        def body(i, carry):
            gs = i * K
            for s in range(K):
                g = gs + s
                drain(g, s)
                for r in range(RPG):
                    lo = _tree_sum([buf_v[s, r * L + l, pl.ds(0, HALF)]
                                    for l in range(L)])
                    hi = _tree_sum([buf_v[s, r * L + l, pl.ds(HALF, HALF)]
                                    for l in range(L)])
                    row = g * RPG + r
                    ocol = (row % 4) * D
                    out_v[row // 4, pl.ds(ocol, HALF)] = lo
                    out_v[row // 4, pl.ds(ocol + HALF, HALF)] = hi

                @pl.when(g + K < ng)
                def _():
                    fire(g + K, s)
            return carry

        lax.fori_loop(0, ng // K, body, 0)
        pltpu.sync_copy(out_v, out_hbm.at[pl.ds(wid * orows_w, orows_w)])

    return run(ids2, wl).reshape(B, D)


# speedup vs baseline: 3.1221x; 3.1221x over previous
import functools

import jax
import jax.numpy as jnp
from jax import lax
from jax.experimental import pallas as pl
from jax.experimental.pallas import tpu as pltpu
from jax.experimental.pallas import tpu_sc as plsc

HALF = 16
RPG = 2
K = 8


def _tree_sum(vals):
    while len(vals) > 1:
        nxt = [vals[i] + vals[i + 1] for i in range(0, len(vals) - 1, 2)]
        if len(vals) % 2:
            nxt.append(vals[-1])
        vals = nxt
    return vals[0]


def _tr_body(in_ref, out_ref):
    eye = jnp.eye(32, dtype=jnp.float32)
    groups = []
    for s in range(8):
        groups.append(jnp.concatenate(
            [jax.lax.dot_general(
                in_ref[:, 512 * s + 128 * c:512 * s + 128 * (c + 1)], eye,
                (((0,), (0,)), ((), ())))
             for c in range(4)], axis=1))
    out_ref[...] = jnp.concatenate(groups, axis=0)


def kernel(ids, weights):
    B, L = ids.shape
    V, D = weights.shape
    info = plsc.get_sparse_core_info()
    nw = info.num_cores * info.num_subcores
    rows_w = B // nw
    idx_per_g = RPG * L
    ng = rows_w // RPG

    nbc = 4096
    nb = (V + nbc - 1) // nbc
    wc = pl.pallas_call(
        _tr_body,
        grid=(nb,),
        in_specs=[pl.BlockSpec((D, nbc), lambda i: (0, i))],
        out_specs=pl.BlockSpec((nbc * D // 128, 128), lambda i: (i, 0)),
        out_shape=jax.ShapeDtypeStruct((nb * nbc * D // 128, 128),
                                       jnp.float32),
    )(weights.T)
    wl = wc.reshape(-1, D)
    idm = ((ids >> 9) << 9) + ((ids & 127) << 2) + ((ids >> 7) & 3)
    ids2 = idm.reshape(B // RPG, idx_per_g)

    mesh = plsc.VectorSubcoreMesh(core_axis_name="c", subcore_axis_name="s")

    @functools.partial(
        pl.kernel,
        mesh=mesh,
        compiler_params=pltpu.CompilerParams(use_tc_tiling_on_sc=False),
        out_type=jax.ShapeDtypeStruct((B * D // 128, 128), jnp.float32),
        scratch_types=[
            pltpu.VMEM((ng, idx_per_g), jnp.int32),
            pltpu.VMEM((K, idx_per_g, D), jnp.float32),
            pltpu.VMEM((rows_w * D // 128, 128), jnp.float32),
        ] + [pltpu.SemaphoreType.DMA] * K,
    )
    def run(ids_hbm, tab_hbm, out_hbm, ids_v, buf_v, out_v, *sems):
        wid = lax.axis_index("s") * info.num_cores + lax.axis_index("c")
        gbase = wid * ng
        orows_w = rows_w * D // 128
        pltpu.sync_copy(ids_hbm.at[pl.ds(gbase, ng)], ids_v)

        def fire(g, s):
            pltpu.async_copy(tab_hbm.at[ids_v.at[g]], buf_v.at[s], sems[s])

        def drain(g, s):
            pltpu.make_async_copy(
                tab_hbm.at[ids_v.at[g]], buf_v.at[s], sems[s]).wait()

        for s in range(K):
            fire(s, s)

        def body(i, carry):
            gs = i * K
            for s in range(K):
                g = gs + s
                drain(g, s)
                for r in range(RPG):
                    lo = _tree_sum([buf_v[s, r * L + l, pl.ds(0, HALF)]
                                    for l in range(L)])
                    hi = _tree_sum([buf_v[s, r * L + l, pl.ds(HALF, HALF)]
                                    for l in range(L)])
                    row = g * RPG + r
                    ocol = (row % 4) * D
                    out_v[row // 4, pl.ds(ocol, HALF)] = lo
                    out_v[row // 4, pl.ds(ocol + HALF, HALF)] = hi

                @pl.when(g + K < ng)
                def _():
                    fire(g + K, s)
            return carry

        lax.fori_loop(0, ng // K, body, 0)
        pltpu.sync_copy(out_v, out_hbm.at[pl.ds(wid * orows_w, orows_w)])

    return run(ids2, wl).reshape(B, D)
